# baseline (speedup 1.0000x reference)
import functools

import jax
import jax.numpy as jnp
from jax import lax
from jax.experimental import pallas as pl
from jax.experimental.pallas import tpu as pltpu
from jax.experimental.pallas import tpu_sc as plsc

VOCAB = 1000000
EMB = 64
B = 16384
CTX = 20

NC, NS = 2, 16
NW = NC * NS
EPW = B // NW
CB = 32
NCHUNK = EPW // CB
ROWS = CB * CTX
IDXW = 128
IDXROWS = ROWS // IDXW
NV = EMB // 16
NBLK = VOCAB // 128
NPAIR = NBLK // 2
KMAX = NPAIR // NW + 1


def _log_sigmoid(x):
    e = jnp.exp(-jnp.abs(x))
    z = e / (e + 2.0)
    z2 = z * z
    p = 1.0 + z2 * ((1.0 / 3.0) + z2 * ((1.0 / 5.0) + z2 * ((1.0 / 7.0) + z2 * (1.0 / 9.0))))
    return jnp.minimum(x, 0.0) - 2.0 * z * p


def _fmt_body(u_t, u_tail, v_t, v_tail, u_scr, v_scr,
              blk0a_v, blk0b_v, blk1a_v, blk1b_v, out0_v, out1_v, rsem, wsem):
    wid = lax.axis_index("s") * NC + lax.axis_index("c")
    lanes = lax.iota(jnp.int32, 16)
    rowvecs = [lanes + 16 * w for w in range(NV)]

    def transpose_rows(blk_v, out_v, qbase, nq):
        def q_body(q2, _):
            for u in range(2):
                q = q2 * 2 + u
                for w in range(8):
                    cvec = jnp.full((16,), 2 * q + w // 4, jnp.int32)
                    out_v[qbase + q, pl.ds(w * 16, 16)] = plsc.load_gather(
                        blk_v, [rowvecs[w % 4], cvec])
            return 0

        lax.fori_loop(0, nq // 2, q_body, 0)

    for src, tail, scr in ((u_t, u_tail, u_scr), (v_t, v_tail, v_scr)):
        bufs = ((blk0a_v, blk0b_v), (blk1a_v, blk1b_v))
        outs = (out0_v, out1_v)
        my_blocks = (NPAIR - 1 - wid) // NW + 1

        def read_blk(k, bufa, bufb):
            cpr = k * NW + wid

            @pl.when(cpr < NPAIR)
            def _():
                pltpu.async_copy(src.at[:, pl.ds(cpr * 256, 128)], bufa, rsem)
                pltpu.async_copy(src.at[:, pl.ds(cpr * 256 + 128, 128)], bufb,
                                 rsem)

        read_blk(0, *bufs[0])

        def k_body(k, _):
            cur = lax.rem(k, 2)
            cpr = k * NW + wid

            @pl.when(cpr < NPAIR)
            def _():
                pltpu.make_async_copy(
                    src.at[:, pl.ds(0, 128)], blk0a_v, rsem).wait()
                pltpu.make_async_copy(
                    src.at[:, pl.ds(0, 128)], blk0a_v, rsem).wait()
                for nxt in range(2):
                    @pl.when(cur == nxt)
                    def _():
                        read_blk(k + 1, *bufs[1 - nxt])

                        @pl.when(k >= 2)
                        def _():
                            pltpu.make_async_copy(
                                out0_v, scr.at[pl.ds(0, 2 * EMB), :], wsem).wait()

                        transpose_rows(bufs[nxt][0], outs[nxt], 0, EMB)
                        transpose_rows(bufs[nxt][1], outs[nxt], EMB, EMB)
                        pltpu.async_copy(
                            outs[nxt], scr.at[pl.ds(cpr * 2 * EMB, 2 * EMB), :],
                            wsem)

            return 0

        lax.fori_loop(0, KMAX, k_body, 0)

        for t in range(1, 3):
            @pl.when(my_blocks >= t)
            def _():
                pltpu.make_async_copy(
                    out0_v, scr.at[pl.ds(0, 2 * EMB), :], wsem).wait()

        @pl.when(wid == NW - 1)
        def _():
            pltpu.async_copy(tail, blk0a_v, rsem).wait()
            transpose_rows(blk0a_v, out0_v, 0, 32)
            pltpu.async_copy(out0_v.at[pl.ds(0, 32), :],
                             scr.at[pl.ds(NBLK * EMB, 32), :], wsem).wait()


def _cbow_body(pos_ctx, pos_tgt, neg_ctx, neg_tgt, u_emb, v_emb, out,
               idx_v, rows_v, tgt_v, vrows_v, usum_v, dots_v, out_v, gsem):
    wid = lax.axis_index("s") * NC + lax.axis_index("c")

    def chunk_body(c, _):
        base = wid * EPW + c * CB

        for side, (ctx_hbm, tgt_hbm) in enumerate(((pos_ctx, pos_tgt), (neg_ctx, neg_tgt))):
            pltpu.sync_copy(ctx_hbm.at[pl.ds(base * CTX, ROWS)], idx_v)
            copies = [
                pltpu.async_copy(u_emb.at[idx_v.at[pl.ds(j * IDXW, IDXW)]],
                                 rows_v.at[pl.ds(j * IDXW, IDXW)], gsem)
                for j in range(IDXROWS)
            ]
            pltpu.sync_copy(tgt_hbm.at[pl.ds(base, CB)], tgt_v)
            vcp = pltpu.async_copy(v_emb.at[tgt_v], vrows_v, gsem)
            for cp in copies:
                cp.wait()
            vcp.wait()

            def elem_body(i, _):
                rbase = i * CTX
                for d in range(NV):
                    acc = rows_v[rbase, pl.ds(d * 16, 16)]
                    for j in range(1, CTX):
                        acc = acc + rows_v[rbase + j, pl.ds(d * 16, 16)]
                    usum_v[i, pl.ds(d * 16, 16)] = acc
                return 0

            lax.fori_loop(0, CB, elem_body, 0)

            lanes = lax.iota(jnp.int32, 16)
            for g in range(CB // 16):
                idx_i = lanes + g * 16

                def dot_body(d, acc):
                    dcol = jnp.full((16,), d, jnp.int32)
                    u_d = plsc.load_gather(usum_v, [idx_i, dcol])
                    v_d = plsc.load_gather(vrows_v, [idx_i, dcol])
                    return acc + u_d * v_d

                dv = lax.fori_loop(0, EMB, dot_body, jnp.zeros((16,), jnp.float32))
                dots_v[side, pl.ds(g * 16, 16)] = dv

        for g in range(CB // 16):
            dp = dots_v[0, pl.ds(g * 16, 16)]
            dn = dots_v[1, pl.ds(g * 16, 16)]
            out_v[pl.ds(g * 16, 16)] = -(_log_sigmoid(dp) + _log_sigmoid(-dn))
        pltpu.sync_copy(out_v, out.at[pl.ds(base, CB)])
        return 0

    lax.fori_loop(0, NCHUNK, chunk_body, 0)


def kernel(pos_context_word_ids, pos_target_word_id,
           neg_context_word_ids, neg_target_word_id, u_emb, v_emb):
    pos_ctx = pos_context_word_ids.reshape(B * CTX)
    neg_ctx = neg_context_word_ids.reshape(B * CTX)
    pos_tgt = pos_target_word_id.reshape(B)
    neg_tgt = neg_target_word_id.reshape(B)
    u_t = u_emb.T
    v_t = v_emb.T
    u_tail = jnp.pad(u_emb[VOCAB - 64:].T, ((0, 0), (0, 64)))
    v_tail = jnp.pad(v_emb[VOCAB - 64:].T, ((0, 0), (0, 64)))

    mesh = plsc.VectorSubcoreMesh(core_axis_name="c", subcore_axis_name="s")
    fmt = functools.partial(
        pl.kernel,
        mesh=mesh,
        compiler_params=pltpu.CompilerParams(
            needs_layout_passes=False, use_tc_tiling_on_sc=True),
        out_type=(jax.ShapeDtypeStruct((VOCAB // 2, 128), jnp.float32),
                  jax.ShapeDtypeStruct((VOCAB // 2, 128), jnp.float32)),
        scratch_types=[
            pltpu.VMEM((EMB, 128), jnp.float32),
            pltpu.VMEM((EMB, 128), jnp.float32),
            pltpu.VMEM((EMB, 128), jnp.float32),
            pltpu.VMEM((EMB, 128), jnp.float32),
            pltpu.VMEM((2 * EMB, 128), jnp.float32),
            pltpu.VMEM((2 * EMB, 128), jnp.float32),
            pltpu.SemaphoreType.DMA,
            pltpu.SemaphoreType.DMA,
        ],
    )(_fmt_body)
    u_lin, v_lin = fmt(u_t, u_tail, v_t, v_tail)

    run = functools.partial(
        pl.kernel,
        mesh=mesh,
        compiler_params=pltpu.CompilerParams(
            needs_layout_passes=False, use_tc_tiling_on_sc=False),
        out_type=jax.ShapeDtypeStruct((B,), jnp.float32),
        scratch_types=[
            pltpu.VMEM((ROWS,), jnp.int32),
            pltpu.VMEM((ROWS, EMB), jnp.float32),
            pltpu.VMEM((CB,), jnp.int32),
            pltpu.VMEM((CB, EMB), jnp.float32),
            pltpu.VMEM((CB, EMB), jnp.float32),
            pltpu.VMEM((2, CB), jnp.float32),
            pltpu.VMEM((CB,), jnp.float32),
            pltpu.SemaphoreType.DMA,
        ],
    )(_cbow_body)
    return run(pos_ctx, pos_tgt, neg_ctx, neg_tgt,
               u_lin.reshape(VOCAB, EMB), v_lin.reshape(VOCAB, EMB))


# speedup vs baseline: 2.8763x; 2.8763x over previous
import functools

import jax
import jax.numpy as jnp
from jax import lax
from jax.experimental import pallas as pl
from jax.experimental.pallas import tpu as pltpu
from jax.experimental.pallas import tpu_sc as plsc

VOCAB = 1000000
EMB = 64
B = 16384
CTX = 20

NC, NS = 2, 16
NW = NC * NS
EPW = B // NW
CB = 32
NCHUNK = EPW // CB
ROWS = CB * CTX
IDXW = 128
IDXROWS = ROWS // IDXW
NV = EMB // 16


def _log_sigmoid(x):
    e = jnp.exp(-jnp.abs(x))
    z = e / (e + 2.0)
    z2 = z * z
    p = 1.0 + z2 * ((1.0 / 3.0) + z2 * ((1.0 / 5.0) + z2 * ((1.0 / 7.0) + z2 * (1.0 / 9.0))))
    return jnp.minimum(x, 0.0) - 2.0 * z * p


def _cbow_body(pos_ctx, pos_vr, neg_ctx, neg_vr, u_emb, out,
               idx_v, rows_v, vrows_v, usum_v, dots_v, out_v, gsem):
    wid = lax.axis_index("s") * NC + lax.axis_index("c")

    def chunk_body(c, _):
        base = wid * EPW + c * CB

        for side, (ctx_hbm, vr_hbm) in enumerate(((pos_ctx, pos_vr), (neg_ctx, neg_vr))):
            pltpu.sync_copy(ctx_hbm.at[pl.ds(base * CTX, ROWS)], idx_v)
            copies = [
                pltpu.async_copy(u_emb.at[idx_v.at[pl.ds(j * IDXW, IDXW)]],
                                 rows_v.at[pl.ds(j * IDXW, IDXW)], gsem)
                for j in range(IDXROWS)
            ]
            vcp = pltpu.async_copy(vr_hbm.at[pl.ds(base, CB), :], vrows_v, gsem)
            for cp in copies:
                cp.wait()
            vcp.wait()

            def elem_body(i, _):
                rbase = i * CTX
                for d in range(NV):
                    acc = rows_v[rbase, pl.ds(d * 16, 16)]
                    for j in range(1, CTX):
                        acc = acc + rows_v[rbase + j, pl.ds(d * 16, 16)]
                    usum_v[i, pl.ds(d * 16, 16)] = acc
                return 0

            lax.fori_loop(0, CB, elem_body, 0)

            lanes = lax.iota(jnp.int32, 16)
            for g in range(CB // 16):
                idx_i = lanes + g * 16

                def dot_body(d, acc):
                    dcol = jnp.full((16,), d, jnp.int32)
                    u_d = plsc.load_gather(usum_v, [idx_i, dcol])
                    v_d = plsc.load_gather(vrows_v, [idx_i, dcol])
                    return acc + u_d * v_d

                dv = lax.fori_loop(0, EMB, dot_body, jnp.zeros((16,), jnp.float32))
                dots_v[side, pl.ds(g * 16, 16)] = dv

        for g in range(CB // 16):
            dp = dots_v[0, pl.ds(g * 16, 16)]
            dn = dots_v[1, pl.ds(g * 16, 16)]
            out_v[pl.ds(g * 16, 16)] = -(_log_sigmoid(dp) + _log_sigmoid(-dn))
        pltpu.sync_copy(out_v, out.at[pl.ds(base, CB)])
        return 0

    lax.fori_loop(0, NCHUNK, chunk_body, 0)


def kernel(pos_context_word_ids, pos_target_word_id,
           neg_context_word_ids, neg_target_word_id, u_emb, v_emb):
    pos_ctx = pos_context_word_ids.reshape(B * CTX)
    neg_ctx = neg_context_word_ids.reshape(B * CTX)
    pos_vr = jnp.take(v_emb, pos_target_word_id.reshape(B), axis=0)
    neg_vr = jnp.take(v_emb, neg_target_word_id.reshape(B), axis=0)

    mesh = plsc.VectorSubcoreMesh(core_axis_name="c", subcore_axis_name="s")
    run = functools.partial(
        pl.kernel,
        mesh=mesh,
        compiler_params=pltpu.CompilerParams(
            needs_layout_passes=False, use_tc_tiling_on_sc=False),
        out_type=jax.ShapeDtypeStruct((B,), jnp.float32),
        scratch_types=[
            pltpu.VMEM((ROWS,), jnp.int32),
            pltpu.VMEM((ROWS, EMB), jnp.float32),
            pltpu.VMEM((CB, EMB), jnp.float32),
            pltpu.VMEM((CB, EMB), jnp.float32),
            pltpu.VMEM((2, CB), jnp.float32),
            pltpu.VMEM((CB,), jnp.float32),
            pltpu.SemaphoreType.DMA,
        ],
    )(_cbow_body)
    return run(pos_ctx, pos_vr, neg_ctx, neg_vr, u_emb)


# speedup vs baseline: 3.0554x; 1.0623x over previous
import functools

import jax
import jax.numpy as jnp
from jax import lax
from jax.experimental import pallas as pl
from jax.experimental.pallas import tpu as pltpu
from jax.experimental.pallas import tpu_sc as plsc

VOCAB = 1000000
EMB = 64
B = 16384
CTX = 20

NC, NS = 2, 16
NW = NC * NS
EPW = B // NW
CB = 32
NCHUNK = EPW // CB
ROWS = CB * CTX
IDXW = 128
IDXROWS = ROWS // IDXW
NV = EMB // 16


def _log_sigmoid(x):
    e = jnp.exp(-jnp.abs(x))
    z = e / (e + 2.0)
    z2 = z * z
    p = 1.0 + z2 * ((1.0 / 3.0) + z2 * ((1.0 / 5.0) + z2 * ((1.0 / 7.0) + z2 * (1.0 / 9.0))))
    return jnp.minimum(x, 0.0) - 2.0 * z * p


def _cbow_body(pos_ctx, pos_vr, neg_ctx, neg_vr, u_emb, out,
               idx0_v, idx1_v, rows0_v, rows1_v, vr0_v, vr1_v,
               usum_v, dots_v, out_v, sem0, sem1):
    wid = lax.axis_index("s") * NC + lax.axis_index("c")
    idxb = (idx0_v, idx1_v)
    rowsb = (rows0_v, rows1_v)
    vrb = (vr0_v, vr1_v)
    sems = (sem0, sem1)

    def fire(ctx_hbm, vr_hbm, c, slot):
        base = wid * EPW + c * CB
        pltpu.sync_copy(ctx_hbm.at[pl.ds(base * CTX, ROWS)], idxb[slot])
        for j in range(IDXROWS):
            pltpu.async_copy(u_emb.at[idxb[slot].at[pl.ds(j * IDXW, IDXW)]],
                             rowsb[slot].at[pl.ds(j * IDXW, IDXW)], sems[slot])
        pltpu.async_copy(vr_hbm.at[pl.ds(base, CB), :], vrb[slot], sems[slot])

    def drain(slot):
        for j in range(IDXROWS):
            pltpu.make_async_copy(
                u_emb.at[idxb[slot].at[pl.ds(0, IDXW)]],
                rowsb[slot].at[pl.ds(0, IDXW)], sems[slot]).wait()
        pltpu.make_async_copy(
            pos_vr.at[pl.ds(0, CB), :], vrb[slot], sems[slot]).wait()

    def compute_side(slot, side):
        rows_v = rowsb[slot]
        vrows_v = vrb[slot]

        def elem_body(i, _):
            rbase = i * CTX
            for d in range(NV):
                acc = rows_v[rbase, pl.ds(d * 16, 16)]
                for j in range(1, CTX):
                    acc = acc + rows_v[rbase + j, pl.ds(d * 16, 16)]
                usum_v[i, pl.ds(d * 16, 16)] = acc
            return 0

        lax.fori_loop(0, CB, elem_body, 0)

        lanes = lax.iota(jnp.int32, 16)
        for g in range(CB // 16):
            idx_i = lanes + g * 16

            def dot_body(d, acc):
                dcol = jnp.full((16,), d, jnp.int32)
                u_d = plsc.load_gather(usum_v, [idx_i, dcol])
                v_d = plsc.load_gather(vrows_v, [idx_i, dcol])
                return acc + u_d * v_d

            dv = lax.fori_loop(0, EMB, dot_body, jnp.zeros((16,), jnp.float32))
            dots_v[side, pl.ds(g * 16, 16)] = dv

    fire(pos_ctx, pos_vr, 0, 0)

    def chunk_body(c, _):
        base = wid * EPW + c * CB
        fire(neg_ctx, neg_vr, c, 1)
        drain(0)
        compute_side(0, 0)

        @pl.when(c + 1 < NCHUNK)
        def _():
            fire(pos_ctx, pos_vr, c + 1, 0)

        drain(1)
        compute_side(1, 1)

        for g in range(CB // 16):
            dp = dots_v[0, pl.ds(g * 16, 16)]
            dn = dots_v[1, pl.ds(g * 16, 16)]
            out_v[pl.ds(g * 16, 16)] = -(_log_sigmoid(dp) + _log_sigmoid(-dn))
        pltpu.sync_copy(out_v, out.at[pl.ds(base, CB)])
        return 0

    lax.fori_loop(0, NCHUNK, chunk_body, 0)


def kernel(pos_context_word_ids, pos_target_word_id,
           neg_context_word_ids, neg_target_word_id, u_emb, v_emb):
    pos_ctx = pos_context_word_ids.reshape(B * CTX)
    neg_ctx = neg_context_word_ids.reshape(B * CTX)
    pos_vr = jnp.take(v_emb, pos_target_word_id.reshape(B), axis=0)
    neg_vr = jnp.take(v_emb, neg_target_word_id.reshape(B), axis=0)

    mesh = plsc.VectorSubcoreMesh(core_axis_name="c", subcore_axis_name="s")
    run = functools.partial(
        pl.kernel,
        mesh=mesh,
        compiler_params=pltpu.CompilerParams(
            needs_layout_passes=False, use_tc_tiling_on_sc=False),
        out_type=jax.ShapeDtypeStruct((B,), jnp.float32),
        scratch_types=[
            pltpu.VMEM((ROWS,), jnp.int32),
            pltpu.VMEM((ROWS,), jnp.int32),
            pltpu.VMEM((ROWS, EMB), jnp.float32),
            pltpu.VMEM((ROWS, EMB), jnp.float32),
            pltpu.VMEM((CB, EMB), jnp.float32),
            pltpu.VMEM((CB, EMB), jnp.float32),
            pltpu.VMEM((CB, EMB), jnp.float32),
            pltpu.VMEM((2, CB), jnp.float32),
            pltpu.VMEM((CB,), jnp.float32),
            pltpu.SemaphoreType.DMA,
            pltpu.SemaphoreType.DMA,
        ],
    )(_cbow_body)
    return run(pos_ctx, pos_vr, neg_ctx, neg_vr, u_emb)
